# Initial kernel scaffold; baseline (speedup 1.0000x reference)
import jax
import jax.numpy as jnp
from jax.experimental import pallas as pl


def kernel(sub, rel, edge_index, edge_type, init_embed, init_rel, pca_W, pca_b, fac_W, rel_W, muW1, mub1, muW2, mub2, lvW1, lvb1, lvW2, lvb2):
    raise NotImplementedError("write your pallas kernel here")


# speedup vs baseline: 23.1441x; 23.1441x over previous
import functools

import jax
import jax.numpy as jnp
from jax import lax
from jax.experimental import pallas as pl
from jax.experimental.pallas import tpu as pltpu
from jax.experimental.pallas import tpu_sc as plsc

N = 10000
E = 160000
K = 3
D = 128
NR = 400
B = 4096
H = 64
NPAIR = K * (K - 1) // 2
KD = K * D
XW = 512

NC = 2
NS = 16
NW = NC * NS

CH = 64
NCH = 80
EPT = NCH * CH
EP = NW * EPT

NP = 10240
BPT = B // NS
ZPT = NP // NS

TC1_ROWS = 1000
TC2_ROWS = 512


def _pca_body(emb_ref, w_ref, b_ref, o_ref):
  t = jnp.tanh(
      jnp.dot(emb_ref[...], w_ref[...], preferred_element_type=jnp.float32)
      + b_ref[...])
  rows = t.shape[0]
  o_ref[...] = jnp.concatenate(
      [t, jnp.ones((rows, 1), jnp.float32),
       jnp.zeros((rows, XW - KD - 1), jnp.float32)], axis=1)


def _sc_body(x_hbm, srcq, dstq, typq, sub_hbm, suboff_hbm, rel_hbm,
             gagg_hbm, xs_hbm, aggf_hbm,
             srcb, typb, eidxb, sub64, w64,
             xr_a, xr_b, rel_a, rel_b,
             sem_x0, sem_x1, sem_r0, sem_r1, sem_w0, sem_w1):
  c = lax.axis_index("c")
  s = lax.axis_index("s")
  wid = c * NS + s

  xr = (xr_a, xr_b)
  rel = (rel_a, rel_b)
  sem_x = (sem_x0, sem_x1)
  sem_r = (sem_r0, sem_r1)
  sem_w = (sem_w0, sem_w1)

  def _zero_bufs(e, _):
    for seg in range(XW // 16):
      xr_a[e, pl.ds(seg * 16, 16)] = jnp.zeros((16,), jnp.float32)
    return 0
  lax.fori_loop(0, CH, _zero_bufs, 0)

  def _zero_stripe(u, _):
    row = c * NP + s * ZPT + u * CH
    pltpu.sync_copy(xr_a, aggf_hbm.at[pl.ds(row, CH)])
    return 0
  lax.fori_loop(0, ZPT // CH, _zero_stripe, 0)
  plsc.subcore_barrier()

  def _issue_gather(h, j):
    pltpu.sync_copy(srcq.at[wid, j], srcb.at[h])
    pltpu.sync_copy(typq.at[wid, j], typb.at[h])
    pltpu.sync_copy(dstq.at[wid, j], eidxb.at[h])
    pltpu.async_copy(x_hbm.at[srcb.at[h]], xr[h], sem_x[h])
    pltpu.async_copy(rel_hbm.at[typb.at[h]], rel[h], sem_r[h])

  def _wait_gather(h):
    pltpu.make_async_copy(x_hbm.at[srcb.at[h]], xr[h], sem_x[h]).wait()
    pltpu.make_async_copy(rel_hbm.at[typb.at[h]], rel[h], sem_r[h]).wait()

  def _mul(h):
    xrh, relh = xr[h], rel[h]

    def _edge(e, _):
      for seg in range(D // 16):
        rl = relh[e, pl.ds(seg * 16, 16)]
        for k in range(K):
          col = k * D + seg * 16
          xrh[e, pl.ds(col, 16)] = xrh[e, pl.ds(col, 16)] * rl
      return 0
    lax.fori_loop(0, CH, _edge, 0)

  def _issue_scatter(h):
    pltpu.async_copy(xr[h], aggf_hbm.at[eidxb.at[h]], sem_w[h], add=True)

  def _wait_scatter(h):
    pltpu.make_async_copy(xr[h], aggf_hbm.at[eidxb.at[h]], sem_w[h]).wait()

  _issue_gather(0, 0)
  _issue_gather(1, 1)

  def _pair(p, _):
    j = p * 2
    _wait_gather(0)
    _mul(0)
    _issue_scatter(0)
    _wait_gather(1)
    _mul(1)
    _issue_scatter(1)
    _wait_scatter(0)

    @pl.when(j + 2 < NCH)
    def _():
      _issue_gather(0, j + 2)
    _wait_scatter(1)

    @pl.when(j + 3 < NCH)
    def _():
      _issue_gather(1, j + 3)
    return 0
  lax.fori_loop(0, NCH // 2, _pair, 0)

  plsc.subcore_barrier()

  for t in range(BPT // CH):
    base = s * BPT + t * CH
    pltpu.sync_copy(suboff_hbm.at[c, pl.ds(base, CH)], w64)
    pltpu.async_copy(aggf_hbm.at[w64], xr_a, sem_x0)
    pltpu.make_async_copy(aggf_hbm.at[w64], xr_a, sem_x0).wait()
    pltpu.sync_copy(xr_a, gagg_hbm.at[c, pl.ds(base, CH)])

    @pl.when(c == 0)
    def _():
      pltpu.sync_copy(sub_hbm.at[pl.ds(base, CH)], sub64)
      pltpu.async_copy(x_hbm.at[sub64], xr_b, sem_x1)
      pltpu.make_async_copy(x_hbm.at[sub64], xr_b, sem_x1).wait()
      pltpu.sync_copy(xr_b, xs_hbm.at[pl.ds(base, CH)])


_sc_call = functools.partial(
    pl.kernel,
    out_type=[
        jax.ShapeDtypeStruct((NC, B, XW), jnp.float32),
        jax.ShapeDtypeStruct((B, XW), jnp.float32),
        jax.ShapeDtypeStruct((NC * NP, XW), jnp.float32),
    ],
    mesh=plsc.VectorSubcoreMesh(
        core_axis_name="c", subcore_axis_name="s", num_cores=NC,
        num_subcores=NS),
    compiler_params=pltpu.CompilerParams(needs_layout_passes=False),
    scratch_types=[
        pltpu.VMEM((2, CH), jnp.int32),
        pltpu.VMEM((2, CH), jnp.int32),
        pltpu.VMEM((2, CH), jnp.int32),
        pltpu.VMEM((CH,), jnp.int32),
        pltpu.VMEM((CH,), jnp.int32),
        pltpu.VMEM((CH, XW), jnp.float32),
        pltpu.VMEM((CH, XW), jnp.float32),
        pltpu.VMEM((CH, D), jnp.float32),
        pltpu.VMEM((CH, D), jnp.float32),
        pltpu.SemaphoreType.DMA,
        pltpu.SemaphoreType.DMA,
        pltpu.SemaphoreType.DMA,
        pltpu.SemaphoreType.DMA,
        pltpu.SemaphoreType.DMA,
        pltpu.SemaphoreType.DMA,
    ],
)(_sc_body)


def _club_body(gagg_ref, xs_ref, facW_ref,
               muW1_ref, mub1_ref, muW2_ref, mub2_ref,
               lvW1_ref, lvb1_ref, lvW2_ref, lvb2_ref, out_ref):
  b = pl.program_id(0)
  acc = gagg_ref[0] + gagg_ref[1]
  deg = acc[:, KD:KD + 1]
  agg = acc[:, :KD] / jnp.maximum(deg, 1.0)
  xs = xs_ref[...]
  x2 = []
  for k in range(K):
    a = jnp.dot(agg[:, k * D:(k + 1) * D], facW_ref[k],
                preferred_element_type=jnp.float32)
    x2.append(jnp.tanh(a + xs[:, k * D:(k + 1) * D]))
  total = jnp.zeros((1, 1), jnp.float32)
  cnt = 0
  for i in range(K):
    for j in range(i + 1, K):
      xa = x2[i]
      ya = x2[j]
      h = jnp.maximum(
          jnp.dot(xa, muW1_ref[cnt], preferred_element_type=jnp.float32)
          + mub1_ref[cnt], 0.0)
      mu = jnp.dot(h, muW2_ref[cnt],
                   preferred_element_type=jnp.float32) + mub2_ref[cnt]
      h2 = jnp.maximum(
          jnp.dot(xa, lvW1_ref[cnt], preferred_element_type=jnp.float32)
          + lvb1_ref[cnt], 0.0)
      lv = jnp.tanh(
          jnp.dot(h2, lvW2_ref[cnt], preferred_element_type=jnp.float32)
          + lvb2_ref[cnt])
      term = (mu - ya) ** 2 * 0.5 * jnp.exp(-lv)
      total = total + jnp.sum(term).reshape(1, 1)
      cnt += 1

  @pl.when(b == 0)
  def _():
    out_ref[...] = jnp.zeros((1, 1), jnp.float32)
  out_ref[...] += total


def kernel(sub, rel, edge_index, edge_type, init_embed, init_rel, pca_W,
           pca_b, fac_W, rel_W, muW1, mub1, muW2, mub2, lvW1, lvb1, lvW2,
           lvb2):
  sub = sub.astype(jnp.int32)
  src = edge_index[0].astype(jnp.int32)
  dst = edge_index[1].astype(jnp.int32)
  typ = edge_type.astype(jnp.int32)

  pad = EP - E
  src_p = jnp.concatenate([src, jnp.zeros((pad,), jnp.int32)])
  dst_p = jnp.concatenate([dst, jnp.full((pad,), N, jnp.int32)])
  typ_p = jnp.concatenate([typ, jnp.zeros((pad,), jnp.int32)])
  src_q = src_p.reshape(NW, NCH, CH)
  typ_q = typ_p.reshape(NW, NCH, CH)
  core_off = (jnp.arange(NW, dtype=jnp.int32) // NS * NP)[:, None, None]
  dst_q = dst_p.reshape(NW, NCH, CH) + core_off
  sub_off = sub[None, :] + (jnp.arange(NC, dtype=jnp.int32) * NP)[:, None]

  x = pl.pallas_call(
      _pca_body,
      grid=(N // TC1_ROWS,),
      in_specs=[
          pl.BlockSpec((TC1_ROWS, D), lambda i: (i, 0)),
          pl.BlockSpec((D, KD), lambda i: (0, 0)),
          pl.BlockSpec((1, KD), lambda i: (0, 0)),
      ],
      out_specs=pl.BlockSpec((TC1_ROWS, XW), lambda i: (i, 0)),
      out_shape=jax.ShapeDtypeStruct((N, XW), jnp.float32),
  )(init_embed, pca_W, pca_b.reshape(1, KD))

  gagg, xs, _ = _sc_call(x, src_q, dst_q, typ_q, sub, sub_off, init_rel)

  nb = B // TC2_ROWS
  loss = pl.pallas_call(
      _club_body,
      grid=(nb,),
      in_specs=[
          pl.BlockSpec((NC, TC2_ROWS, XW), lambda i: (0, i, 0)),
          pl.BlockSpec((TC2_ROWS, XW), lambda i: (i, 0)),
          pl.BlockSpec((K, D, D), lambda i: (0, 0, 0)),
          pl.BlockSpec((NPAIR, D, H), lambda i: (0, 0, 0)),
          pl.BlockSpec((NPAIR, H), lambda i: (0, 0)),
          pl.BlockSpec((NPAIR, H, D), lambda i: (0, 0, 0)),
          pl.BlockSpec((NPAIR, D), lambda i: (0, 0)),
          pl.BlockSpec((NPAIR, D, H), lambda i: (0, 0, 0)),
          pl.BlockSpec((NPAIR, H), lambda i: (0, 0)),
          pl.BlockSpec((NPAIR, H, D), lambda i: (0, 0, 0)),
          pl.BlockSpec((NPAIR, D), lambda i: (0, 0)),
      ],
      out_specs=pl.BlockSpec((1, 1), lambda i: (0, 0)),
      out_shape=jax.ShapeDtypeStruct((1, 1), jnp.float32),
  )(gagg, xs, fac_W, muW1, mub1, muW2, mub2, lvW1, lvb1, lvW2, lvb2)

  return loss[0, 0] / B


# speedup vs baseline: 24.7314x; 1.0686x over previous
import functools

import jax
import jax.numpy as jnp
from jax import lax
from jax.experimental import pallas as pl
from jax.experimental.pallas import tpu as pltpu
from jax.experimental.pallas import tpu_sc as plsc

N = 10000
E = 160000
K = 3
D = 128
NR = 400
B = 4096
H = 64
NPAIR = K * (K - 1) // 2
KD = K * D
XW = 512

NC = 2
NS = 16
NW = NC * NS

CH = 64
NCH = 80
EPT = NCH * CH
EP = NW * EPT

NP = 10240
BPT = B // NS
ZPT = NP // NS

TC1_ROWS = 1000
TC2_ROWS = 512


def _pca_body(emb_ref, w_ref, b_ref, o_ref):
  t = jnp.tanh(
      jnp.dot(emb_ref[...], w_ref[...], preferred_element_type=jnp.float32)
      + b_ref[...])
  rows = t.shape[0]
  o_ref[...] = jnp.concatenate(
      [t, jnp.ones((rows, 1), jnp.float32),
       jnp.zeros((rows, XW - KD - 1), jnp.float32)], axis=1)


def _sc_body(x_hbm, srcq, dstq, typq, sub_hbm, suboff_hbm, rel_hbm,
             gagg_hbm, xs_hbm, aggf_hbm,
             srcb, typb, eidxb, sub64, w64,
             xr_a, xr_b, rel_a, rel_b,
             sem_x0, sem_x1, sem_r0, sem_r1, sem_w0, sem_w1):
  c = lax.axis_index("c")
  s = lax.axis_index("s")
  wid = c * NS + s

  xr = (xr_a, xr_b)
  rel = (rel_a, rel_b)
  sem_x = (sem_x0, sem_x1)
  sem_r = (sem_r0, sem_r1)
  sem_w = (sem_w0, sem_w1)

  def _zero_bufs(e, _):
    for seg in range(XW // 16):
      xr_a[e, pl.ds(seg * 16, 16)] = jnp.zeros((16,), jnp.float32)
    return 0
  lax.fori_loop(0, CH, _zero_bufs, 0)

  def _zero_stripe(u, _):
    row = c * NP + s * ZPT + u * CH
    pltpu.async_copy(xr_a, aggf_hbm.at[pl.ds(row, CH)], sem_w0)
    return 0
  lax.fori_loop(0, ZPT // CH, _zero_stripe, 0)

  def _zero_wait(u, _):
    row = c * NP + s * ZPT + u * CH
    pltpu.make_async_copy(xr_a, aggf_hbm.at[pl.ds(row, CH)], sem_w0).wait()
    return 0
  lax.fori_loop(0, ZPT // CH, _zero_wait, 0)
  plsc.subcore_barrier()

  pltpu.sync_copy(srcq.at[wid], srcb)
  pltpu.sync_copy(typq.at[wid], typb)
  pltpu.sync_copy(dstq.at[wid], eidxb)

  def _issue_gather(h, j):
    pltpu.async_copy(x_hbm.at[srcb.at[j]], xr[h], sem_x[h])
    pltpu.async_copy(rel_hbm.at[typb.at[j]], rel[h], sem_r[h])

  def _wait_gather(h, j):
    pltpu.make_async_copy(x_hbm.at[srcb.at[j]], xr[h], sem_x[h]).wait()
    pltpu.make_async_copy(rel_hbm.at[typb.at[j]], rel[h], sem_r[h]).wait()

  def _mul(h):
    xrh, relh = xr[h], rel[h]

    def _edge(e, _):
      for seg in range(D // 16):
        rl = relh[e, pl.ds(seg * 16, 16)]
        for k in range(K):
          col = k * D + seg * 16
          xrh[e, pl.ds(col, 16)] = xrh[e, pl.ds(col, 16)] * rl
      return 0
    lax.fori_loop(0, CH, _edge, 0)

  def _issue_scatter(h, j):
    pltpu.async_copy(xr[h], aggf_hbm.at[eidxb.at[j]], sem_w[h], add=True)

  def _wait_scatter(h, j):
    pltpu.make_async_copy(xr[h], aggf_hbm.at[eidxb.at[j]], sem_w[h]).wait()

  _issue_gather(0, 0)
  _issue_gather(1, 1)

  def _pair(p, _):
    j = p * 2
    _wait_gather(0, j)
    _mul(0)
    _issue_scatter(0, j)
    _wait_gather(1, j + 1)
    _mul(1)
    _issue_scatter(1, j + 1)
    _wait_scatter(0, j)

    @pl.when(j + 2 < NCH)
    def _():
      _issue_gather(0, j + 2)
    _wait_scatter(1, j + 1)

    @pl.when(j + 3 < NCH)
    def _():
      _issue_gather(1, j + 3)
    return 0
  lax.fori_loop(0, NCH // 2, _pair, 0)

  plsc.subcore_barrier()

  for t in range(BPT // CH):
    base = s * BPT + t * CH
    pltpu.sync_copy(suboff_hbm.at[c, pl.ds(base, CH)], w64)
    pltpu.async_copy(aggf_hbm.at[w64], xr_a, sem_x0)
    pltpu.make_async_copy(aggf_hbm.at[w64], xr_a, sem_x0).wait()
    pltpu.sync_copy(xr_a, gagg_hbm.at[c, pl.ds(base, CH)])

    @pl.when(c == 0)
    def _():
      pltpu.sync_copy(sub_hbm.at[pl.ds(base, CH)], sub64)
      pltpu.async_copy(x_hbm.at[sub64], xr_b, sem_x1)
      pltpu.make_async_copy(x_hbm.at[sub64], xr_b, sem_x1).wait()
      pltpu.sync_copy(xr_b, xs_hbm.at[pl.ds(base, CH)])


_sc_call = functools.partial(
    pl.kernel,
    out_type=[
        jax.ShapeDtypeStruct((NC, B, XW), jnp.float32),
        jax.ShapeDtypeStruct((B, XW), jnp.float32),
        jax.ShapeDtypeStruct((NC * NP, XW), jnp.float32),
    ],
    mesh=plsc.VectorSubcoreMesh(
        core_axis_name="c", subcore_axis_name="s", num_cores=NC,
        num_subcores=NS),
    compiler_params=pltpu.CompilerParams(needs_layout_passes=False),
    scratch_types=[
        pltpu.VMEM((NCH, CH), jnp.int32),
        pltpu.VMEM((NCH, CH), jnp.int32),
        pltpu.VMEM((NCH, CH), jnp.int32),
        pltpu.VMEM((CH,), jnp.int32),
        pltpu.VMEM((CH,), jnp.int32),
        pltpu.VMEM((CH, XW), jnp.float32),
        pltpu.VMEM((CH, XW), jnp.float32),
        pltpu.VMEM((CH, D), jnp.float32),
        pltpu.VMEM((CH, D), jnp.float32),
        pltpu.SemaphoreType.DMA,
        pltpu.SemaphoreType.DMA,
        pltpu.SemaphoreType.DMA,
        pltpu.SemaphoreType.DMA,
        pltpu.SemaphoreType.DMA,
        pltpu.SemaphoreType.DMA,
    ],
)(_sc_body)


def _club_body(gagg_ref, xs_ref, facW_ref,
               muW1_ref, mub1_ref, muW2_ref, mub2_ref,
               lvW1_ref, lvb1_ref, lvW2_ref, lvb2_ref, out_ref):
  b = pl.program_id(0)
  acc = gagg_ref[0] + gagg_ref[1]
  deg = acc[:, KD:KD + 1]
  agg = acc[:, :KD] / jnp.maximum(deg, 1.0)
  xs = xs_ref[...]
  x2 = []
  for k in range(K):
    a = jnp.dot(agg[:, k * D:(k + 1) * D], facW_ref[k],
                preferred_element_type=jnp.float32)
    x2.append(jnp.tanh(a + xs[:, k * D:(k + 1) * D]))
  total = jnp.zeros((1, 1), jnp.float32)
  cnt = 0
  for i in range(K):
    for j in range(i + 1, K):
      xa = x2[i]
      ya = x2[j]
      h = jnp.maximum(
          jnp.dot(xa, muW1_ref[cnt], preferred_element_type=jnp.float32)
          + mub1_ref[cnt], 0.0)
      mu = jnp.dot(h, muW2_ref[cnt],
                   preferred_element_type=jnp.float32) + mub2_ref[cnt]
      h2 = jnp.maximum(
          jnp.dot(xa, lvW1_ref[cnt], preferred_element_type=jnp.float32)
          + lvb1_ref[cnt], 0.0)
      lv = jnp.tanh(
          jnp.dot(h2, lvW2_ref[cnt], preferred_element_type=jnp.float32)
          + lvb2_ref[cnt])
      term = (mu - ya) ** 2 * 0.5 * jnp.exp(-lv)
      total = total + jnp.sum(term).reshape(1, 1)
      cnt += 1

  @pl.when(b == 0)
  def _():
    out_ref[...] = jnp.zeros((1, 1), jnp.float32)
  out_ref[...] += total


def kernel(sub, rel, edge_index, edge_type, init_embed, init_rel, pca_W,
           pca_b, fac_W, rel_W, muW1, mub1, muW2, mub2, lvW1, lvb1, lvW2,
           lvb2):
  sub = sub.astype(jnp.int32)
  src = edge_index[0].astype(jnp.int32)
  dst = edge_index[1].astype(jnp.int32)
  typ = edge_type.astype(jnp.int32)

  pad = EP - E
  src_p = jnp.concatenate([src, jnp.zeros((pad,), jnp.int32)])
  dst_p = jnp.concatenate([dst, jnp.full((pad,), N, jnp.int32)])
  typ_p = jnp.concatenate([typ, jnp.zeros((pad,), jnp.int32)])
  src_q = src_p.reshape(NW, NCH, CH)
  typ_q = typ_p.reshape(NW, NCH, CH)
  core_off = (jnp.arange(NW, dtype=jnp.int32) // NS * NP)[:, None, None]
  dst_q = dst_p.reshape(NW, NCH, CH) + core_off
  sub_off = sub[None, :] + (jnp.arange(NC, dtype=jnp.int32) * NP)[:, None]

  x = pl.pallas_call(
      _pca_body,
      grid=(N // TC1_ROWS,),
      in_specs=[
          pl.BlockSpec((TC1_ROWS, D), lambda i: (i, 0)),
          pl.BlockSpec((D, KD), lambda i: (0, 0)),
          pl.BlockSpec((1, KD), lambda i: (0, 0)),
      ],
      out_specs=pl.BlockSpec((TC1_ROWS, XW), lambda i: (i, 0)),
      out_shape=jax.ShapeDtypeStruct((N, XW), jnp.float32),
  )(init_embed, pca_W, pca_b.reshape(1, KD))

  gagg, xs, _ = _sc_call(x, src_q, dst_q, typ_q, sub, sub_off, init_rel)

  nb = B // TC2_ROWS
  loss = pl.pallas_call(
      _club_body,
      grid=(nb,),
      in_specs=[
          pl.BlockSpec((NC, TC2_ROWS, XW), lambda i: (0, i, 0)),
          pl.BlockSpec((TC2_ROWS, XW), lambda i: (i, 0)),
          pl.BlockSpec((K, D, D), lambda i: (0, 0, 0)),
          pl.BlockSpec((NPAIR, D, H), lambda i: (0, 0, 0)),
          pl.BlockSpec((NPAIR, H), lambda i: (0, 0)),
          pl.BlockSpec((NPAIR, H, D), lambda i: (0, 0, 0)),
          pl.BlockSpec((NPAIR, D), lambda i: (0, 0)),
          pl.BlockSpec((NPAIR, D, H), lambda i: (0, 0, 0)),
          pl.BlockSpec((NPAIR, H), lambda i: (0, 0)),
          pl.BlockSpec((NPAIR, H, D), lambda i: (0, 0, 0)),
          pl.BlockSpec((NPAIR, D), lambda i: (0, 0)),
      ],
      out_specs=pl.BlockSpec((1, 1), lambda i: (0, 0)),
      out_shape=jax.ShapeDtypeStruct((1, 1), jnp.float32),
  )(gagg, xs, fac_W, muW1, mub1, muW2, mub2, lvW1, lvb1, lvW2, lvb2)

  return loss[0, 0] / B


# speedup vs baseline: 29.8304x; 1.2062x over previous
import functools

import jax
import jax.numpy as jnp
from jax import lax
from jax.experimental import pallas as pl
from jax.experimental.pallas import tpu as pltpu
from jax.experimental.pallas import tpu_sc as plsc

N = 10000
E = 160000
K = 3
D = 128
NR = 400
B = 4096
H = 64
NPAIR = K * (K - 1) // 2
KD = K * D
XW = 512

NC = 2
NS = 16
NW = NC * NS

CH = 32
NB = 4
NCH = 160
EPT = NCH * CH
EP = NW * EPT

NP = 10240
BPT = B // NS
ZPT = NP // NS

TC1_ROWS = 1000
TC2_ROWS = 512


def _pca_body(emb_ref, w_ref, b_ref, o_ref):
  o_ref[...] = jnp.tanh(
      jnp.dot(emb_ref[...], w_ref[...], preferred_element_type=jnp.float32)
      + b_ref[...])


def _sc_body(x_hbm, srcq, dstq, typq, sub_hbm, suboff_hbm, rel_hbm,
             gagg_hbm, xs_hbm, aggf_hbm,
             srcb, typb, eidxb, sub32, w32, xsbuf,
             xr_0, xr_1, xr_2, xr_3, rel_0, rel_1, rel_2, rel_3,
             sem_x0, sem_x1, sem_x2, sem_x3,
             sem_r0, sem_r1, sem_r2, sem_r3,
             sem_w0, sem_w1, sem_w2, sem_w3):
  c = lax.axis_index("c")
  s = lax.axis_index("s")
  wid = c * NS + s

  xr = (xr_0, xr_1, xr_2, xr_3)
  rel = (rel_0, rel_1, rel_2, rel_3)
  sem_x = (sem_x0, sem_x1, sem_x2, sem_x3)
  sem_r = (sem_r0, sem_r1, sem_r2, sem_r3)
  sem_w = (sem_w0, sem_w1, sem_w2, sem_w3)

  def _zero_bufs(e, _):
    for seg in range(XW // 16):
      xr_0[e, pl.ds(seg * 16, 16)] = jnp.zeros((16,), jnp.float32)
    return 0
  lax.fori_loop(0, CH, _zero_bufs, 0)

  def _zero_stripe(u, _):
    row = c * NP + s * ZPT + u * CH
    pltpu.async_copy(xr_0, aggf_hbm.at[pl.ds(row, CH)], sem_w0)
    return 0
  lax.fori_loop(0, ZPT // CH, _zero_stripe, 0)

  def _zero_wait(u, _):
    row = c * NP + s * ZPT + u * CH
    pltpu.make_async_copy(xr_0, aggf_hbm.at[pl.ds(row, CH)], sem_w0).wait()
    return 0
  lax.fori_loop(0, ZPT // CH, _zero_wait, 0)

  one0 = jnp.where(lax.iota(jnp.int32, 16) == 0,
                   jnp.ones((16,), jnp.float32),
                   jnp.zeros((16,), jnp.float32))

  def _init_tail(e, _):
    for h in range(NB):
      xr[h][e, pl.ds(KD, 16)] = one0
      for seg in range(KD // 16 + 1, XW // 16):
        xr[h][e, pl.ds(seg * 16, 16)] = jnp.zeros((16,), jnp.float32)
    return 0
  lax.fori_loop(0, CH, _init_tail, 0)
  plsc.subcore_barrier()

  pltpu.sync_copy(srcq.at[wid], srcb)
  pltpu.sync_copy(typq.at[wid], typb)
  pltpu.sync_copy(dstq.at[wid], eidxb)

  def _issue_gather(h, j):
    pltpu.async_copy(x_hbm.at[srcb.at[pl.ds(j * CH, CH)]],
                     xr[h].at[:, pl.ds(0, KD)], sem_x[h])
    pltpu.async_copy(rel_hbm.at[typb.at[pl.ds(j * CH, CH)]], rel[h],
                     sem_r[h])

  def _wait_gather(h, j):
    pltpu.make_async_copy(x_hbm.at[srcb.at[pl.ds(j * CH, CH)]],
                          xr[h].at[:, pl.ds(0, KD)], sem_x[h]).wait()
    pltpu.make_async_copy(rel_hbm.at[typb.at[pl.ds(j * CH, CH)]], rel[h],
                          sem_r[h]).wait()

  def _mul(h):
    xrh, relh = xr[h], rel[h]

    def _edge(e, _):
      for seg in range(D // 16):
        rl = relh[e, pl.ds(seg * 16, 16)]
        for k in range(K):
          col = k * D + seg * 16
          xrh[e, pl.ds(col, 16)] = xrh[e, pl.ds(col, 16)] * rl
      return 0
    lax.fori_loop(0, CH, _edge, 0)

  def _issue_scatter(h, j):
    pltpu.async_copy(xr[h], aggf_hbm.at[eidxb.at[j]], sem_w[h], add=True)

  def _wait_scatter(h, j):
    pltpu.make_async_copy(xr[h], aggf_hbm.at[eidxb.at[j]], sem_w[h]).wait()

  _issue_gather(0, 0)
  _issue_gather(1, 1)

  def _quad(q, _):
    for h in range(NB):
      j = q * NB + h
      _wait_gather(h, j)
      _mul(h)
      _issue_scatter(h, j)
      h2 = (h + 2) % NB

      @pl.when(j >= 2)
      def _():
        _wait_scatter(h2, j - 2)

      @pl.when(j + 2 < NCH)
      def _():
        _issue_gather(h2, j + 2)
    return 0
  lax.fori_loop(0, NCH // NB, _quad, 0)

  _wait_scatter(2, NCH - 2)
  _wait_scatter(3, NCH - 1)
  plsc.subcore_barrier()

  for t in range(BPT // CH):
    base = s * BPT + t * CH
    pltpu.sync_copy(suboff_hbm.at[c, pl.ds(base, CH)], w32)
    pltpu.async_copy(aggf_hbm.at[w32], xr_0, sem_x0)
    pltpu.make_async_copy(aggf_hbm.at[w32], xr_0, sem_x0).wait()
    pltpu.sync_copy(xr_0, gagg_hbm.at[c, pl.ds(base, CH)])

    @pl.when(c == 0)
    def _():
      pltpu.sync_copy(sub_hbm.at[pl.ds(base, CH)], sub32)
      pltpu.async_copy(x_hbm.at[sub32], xsbuf, sem_x1)
      pltpu.make_async_copy(x_hbm.at[sub32], xsbuf, sem_x1).wait()
      pltpu.sync_copy(xsbuf, xs_hbm.at[pl.ds(base, CH)])


_sc_call = functools.partial(
    pl.kernel,
    out_type=[
        jax.ShapeDtypeStruct((NC, B, XW), jnp.float32),
        jax.ShapeDtypeStruct((B, KD), jnp.float32),
        jax.ShapeDtypeStruct((NC * NP, XW), jnp.float32),
    ],
    mesh=plsc.VectorSubcoreMesh(
        core_axis_name="c", subcore_axis_name="s", num_cores=NC,
        num_subcores=NS),
    compiler_params=pltpu.CompilerParams(needs_layout_passes=False),
    scratch_types=[
        pltpu.VMEM((EPT,), jnp.int32),
        pltpu.VMEM((EPT,), jnp.int32),
        pltpu.VMEM((NCH, CH), jnp.int32),
        pltpu.VMEM((CH,), jnp.int32),
        pltpu.VMEM((CH,), jnp.int32),
        pltpu.VMEM((CH, KD), jnp.float32),
        pltpu.VMEM((CH, XW), jnp.float32),
        pltpu.VMEM((CH, XW), jnp.float32),
        pltpu.VMEM((CH, XW), jnp.float32),
        pltpu.VMEM((CH, XW), jnp.float32),
        pltpu.VMEM((CH, D), jnp.float32),
        pltpu.VMEM((CH, D), jnp.float32),
        pltpu.VMEM((CH, D), jnp.float32),
        pltpu.VMEM((CH, D), jnp.float32),
    ] + [pltpu.SemaphoreType.DMA] * 12,
)(_sc_body)


def _club_body(gagg_ref, xs_ref, facW_ref,
               muW1_ref, mub1_ref, muW2_ref, mub2_ref,
               lvW1_ref, lvb1_ref, lvW2_ref, lvb2_ref, out_ref):
  b = pl.program_id(0)
  acc = gagg_ref[0] + gagg_ref[1]
  deg = acc[:, KD:KD + 1]
  agg = acc[:, :KD] / jnp.maximum(deg, 1.0)
  xs = xs_ref[...]
  x2 = []
  for k in range(K):
    a = jnp.dot(agg[:, k * D:(k + 1) * D], facW_ref[k],
                preferred_element_type=jnp.float32)
    x2.append(jnp.tanh(a + xs[:, k * D:(k + 1) * D]))
  total = jnp.zeros((1, 1), jnp.float32)
  cnt = 0
  for i in range(K):
    for j in range(i + 1, K):
      xa = x2[i]
      ya = x2[j]
      h = jnp.maximum(
          jnp.dot(xa, muW1_ref[cnt], preferred_element_type=jnp.float32)
          + mub1_ref[cnt], 0.0)
      mu = jnp.dot(h, muW2_ref[cnt],
                   preferred_element_type=jnp.float32) + mub2_ref[cnt]
      h2 = jnp.maximum(
          jnp.dot(xa, lvW1_ref[cnt], preferred_element_type=jnp.float32)
          + lvb1_ref[cnt], 0.0)
      lv = jnp.tanh(
          jnp.dot(h2, lvW2_ref[cnt], preferred_element_type=jnp.float32)
          + lvb2_ref[cnt])
      term = (mu - ya) ** 2 * 0.5 * jnp.exp(-lv)
      total = total + jnp.sum(term).reshape(1, 1)
      cnt += 1

  @pl.when(b == 0)
  def _():
    out_ref[...] = jnp.zeros((1, 1), jnp.float32)
  out_ref[...] += total


def kernel(sub, rel, edge_index, edge_type, init_embed, init_rel, pca_W,
           pca_b, fac_W, rel_W, muW1, mub1, muW2, mub2, lvW1, lvb1, lvW2,
           lvb2):
  sub = sub.astype(jnp.int32)
  src = edge_index[0].astype(jnp.int32)
  dst = edge_index[1].astype(jnp.int32)
  typ = edge_type.astype(jnp.int32)

  pad = EP - E
  src_p = jnp.concatenate([src, jnp.zeros((pad,), jnp.int32)])
  dst_p = jnp.concatenate([dst, jnp.full((pad,), N, jnp.int32)])
  typ_p = jnp.concatenate([typ, jnp.zeros((pad,), jnp.int32)])
  src_q = src_p.reshape(NW, EPT)
  typ_q = typ_p.reshape(NW, EPT)
  core_off = (jnp.arange(NW, dtype=jnp.int32) // NS * NP)[:, None, None]
  dst_q = dst_p.reshape(NW, NCH, CH) + core_off
  sub_off = sub[None, :] + (jnp.arange(NC, dtype=jnp.int32) * NP)[:, None]

  x = pl.pallas_call(
      _pca_body,
      grid=(N // TC1_ROWS,),
      in_specs=[
          pl.BlockSpec((TC1_ROWS, D), lambda i: (i, 0)),
          pl.BlockSpec((D, KD), lambda i: (0, 0)),
          pl.BlockSpec((1, KD), lambda i: (0, 0)),
      ],
      out_specs=pl.BlockSpec((TC1_ROWS, KD), lambda i: (i, 0)),
      out_shape=jax.ShapeDtypeStruct((N, KD), jnp.float32),
  )(init_embed, pca_W, pca_b.reshape(1, KD))

  gagg, xs, _ = _sc_call(x, src_q, dst_q, typ_q, sub, sub_off, init_rel)

  nb = B // TC2_ROWS
  loss = pl.pallas_call(
      _club_body,
      grid=(nb,),
      in_specs=[
          pl.BlockSpec((NC, TC2_ROWS, XW), lambda i: (0, i, 0)),
          pl.BlockSpec((TC2_ROWS, KD), lambda i: (i, 0)),
          pl.BlockSpec((K, D, D), lambda i: (0, 0, 0)),
          pl.BlockSpec((NPAIR, D, H), lambda i: (0, 0, 0)),
          pl.BlockSpec((NPAIR, H), lambda i: (0, 0)),
          pl.BlockSpec((NPAIR, H, D), lambda i: (0, 0, 0)),
          pl.BlockSpec((NPAIR, D), lambda i: (0, 0)),
          pl.BlockSpec((NPAIR, D, H), lambda i: (0, 0, 0)),
          pl.BlockSpec((NPAIR, H), lambda i: (0, 0)),
          pl.BlockSpec((NPAIR, H, D), lambda i: (0, 0, 0)),
          pl.BlockSpec((NPAIR, D), lambda i: (0, 0)),
      ],
      out_specs=pl.BlockSpec((1, 1), lambda i: (0, 0)),
      out_shape=jax.ShapeDtypeStruct((1, 1), jnp.float32),
  )(gagg, xs, fac_W, muW1, mub1, muW2, mub2, lvW1, lvb1, lvW2, lvb2)

  return loss[0, 0] / B
